# baseline (speedup 1.0000x reference)
import functools

import jax
import jax.numpy as jnp
from jax import lax
from jax.experimental import pallas as pl
from jax.experimental.pallas import tpu as pltpu
from jax.experimental.pallas import tpu_sc as plsc

NUM_EMB = 1000000
D = 32
NSAMP = 16384
SEQ = 200

NC, NS = 2, 16
NW = NC * NS
SAMP_PER_CHUNK = 8
SAMP_PER_W = NSAMP // NW
N_CHUNK = SAMP_PER_W // SAMP_PER_CHUNK

_mesh = plsc.VectorSubcoreMesh(core_axis_name="c", subcore_axis_name="s")


@functools.partial(
    pl.kernel,
    mesh=_mesh,
    out_type=jax.ShapeDtypeStruct((NSAMP, SEQ, D), jnp.float32),
    scratch_types=[
        pltpu.VMEM((2, SAMP_PER_CHUNK, SEQ), jnp.int32),
        pltpu.VMEM((2, SAMP_PER_CHUNK, SEQ, D), jnp.float32),
        pltpu.SemaphoreType.DMA,
        pltpu.SemaphoreType.DMA,
        pltpu.SemaphoreType.DMA,
        pltpu.SemaphoreType.DMA,
    ],
    compiler_params=pltpu.CompilerParams(use_tc_tiling_on_sc=False),
)
def _emb_lookup(idx_hbm, table_hbm, out_hbm, idx_v, rows_v,
                sem_idx, sem_g, sem_out0, sem_out1):
    wid = lax.axis_index("s") * NC + lax.axis_index("c")
    samp0 = wid * SAMP_PER_W
    sem_out = (sem_out0, sem_out1)

    pltpu.async_copy(idx_hbm.at[pl.ds(samp0, SAMP_PER_CHUNK)], idx_v.at[0],
                     sem_idx)

    def pair_body(g, carry):
        for b in range(2):
            c = 2 * g + b
            s = samp0 + c * SAMP_PER_CHUNK

            @pl.when(g > 0)
            def _():
                pltpu.make_async_copy(
                    rows_v.at[b], out_hbm.at[pl.ds(s, SAMP_PER_CHUNK)],
                    sem_out[b]).wait()

            pltpu.make_async_copy(
                idx_hbm.at[pl.ds(s, SAMP_PER_CHUNK)], idx_v.at[b],
                sem_idx).wait()

            gathers = [
                pltpu.async_copy(table_hbm.at[idx_v.at[b].at[j]],
                                 rows_v.at[b].at[j], sem_g)
                for j in range(SAMP_PER_CHUNK)
            ]

            ns = samp0 + jnp.minimum(c + 1, N_CHUNK - 1) * SAMP_PER_CHUNK
            pltpu.async_copy(idx_hbm.at[pl.ds(ns, SAMP_PER_CHUNK)],
                             idx_v.at[1 - b], sem_idx)

            for gth in gathers:
                gth.wait()
            pltpu.async_copy(rows_v.at[b],
                             out_hbm.at[pl.ds(s, SAMP_PER_CHUNK)], sem_out[b])
        return carry

    lax.fori_loop(0, N_CHUNK // 2, pair_body, 0)

    pltpu.make_async_copy(idx_hbm.at[pl.ds(samp0, SAMP_PER_CHUNK)],
                          idx_v.at[0], sem_idx).wait()
    for b in range(2):
        tail = samp0 + (N_CHUNK - 2 + b) * SAMP_PER_CHUNK
        pltpu.make_async_copy(rows_v.at[b],
                              out_hbm.at[pl.ds(tail, SAMP_PER_CHUNK)],
                              sem_out[b]).wait()


def kernel(input_ids, table):
    return _emb_lookup(input_ids, table)


# speedup vs baseline: 1.0028x; 1.0028x over previous
import functools

import jax
import jax.numpy as jnp
from jax import lax
from jax.experimental import pallas as pl
from jax.experimental.pallas import tpu as pltpu
from jax.experimental.pallas import tpu_sc as plsc

NUM_EMB = 1000000
D = 32
NSAMP = 16384
SEQ = 200

NC, NS = 2, 16
NW = NC * NS
SAMP_PER_CHUNK = 8
SAMP_PER_W = NSAMP // NW
N_CHUNK = SAMP_PER_W // SAMP_PER_CHUNK

_mesh = plsc.VectorSubcoreMesh(core_axis_name="c", subcore_axis_name="s")


@functools.partial(
    pl.kernel,
    mesh=_mesh,
    out_type=jax.ShapeDtypeStruct((NSAMP, SEQ, D), jnp.float32),
    scratch_types=[
        pltpu.VMEM((2, SAMP_PER_CHUNK, SEQ), jnp.int32),
        pltpu.VMEM((2, SAMP_PER_CHUNK, SEQ, D), jnp.float32),
        pltpu.SemaphoreType.DMA,
        pltpu.SemaphoreType.DMA,
        pltpu.SemaphoreType.DMA,
        pltpu.SemaphoreType.DMA,
        pltpu.SemaphoreType.DMA,
    ],
    compiler_params=pltpu.CompilerParams(use_tc_tiling_on_sc=False),
)
def _emb_lookup(idx_hbm, table_hbm, out_hbm, idx_v, rows_v,
                sem_idx, sem_g0, sem_g1, sem_out0, sem_out1):
    wid = lax.axis_index("s") * NC + lax.axis_index("c")
    samp0 = wid * SAMP_PER_W
    sem_g = (sem_g0, sem_g1)
    sem_out = (sem_out0, sem_out1)

    def fire_gathers(b):
        for j in range(SAMP_PER_CHUNK):
            pltpu.async_copy(table_hbm.at[idx_v.at[b].at[j]],
                             rows_v.at[b].at[j], sem_g[b])

    def finish_chunk(b, s):
        for j in range(SAMP_PER_CHUNK):
            pltpu.make_async_copy(table_hbm.at[idx_v.at[b].at[j]],
                                  rows_v.at[b].at[j], sem_g[b]).wait()
        pltpu.async_copy(rows_v.at[b], out_hbm.at[pl.ds(s, SAMP_PER_CHUNK)],
                         sem_out[b])

    pltpu.async_copy(idx_hbm.at[pl.ds(samp0, SAMP_PER_CHUNK)], idx_v.at[0],
                     sem_idx)

    def pair_body(g, carry):
        for b in range(2):
            c = 2 * g + b
            s = samp0 + c * SAMP_PER_CHUNK

            @pl.when(g > 0)
            def _():
                pltpu.make_async_copy(
                    rows_v.at[b], out_hbm.at[pl.ds(s, SAMP_PER_CHUNK)],
                    sem_out[b]).wait()

            pltpu.make_async_copy(
                idx_hbm.at[pl.ds(s, SAMP_PER_CHUNK)], idx_v.at[b],
                sem_idx).wait()

            fire_gathers(b)

            if b == 1:
                finish_chunk(0, s - SAMP_PER_CHUNK)
            else:
                @pl.when(g > 0)
                def _():
                    finish_chunk(1, s - SAMP_PER_CHUNK)

            ns = samp0 + jnp.minimum(c + 1, N_CHUNK - 1) * SAMP_PER_CHUNK
            pltpu.async_copy(idx_hbm.at[pl.ds(ns, SAMP_PER_CHUNK)],
                             idx_v.at[1 - b], sem_idx)
        return carry

    lax.fori_loop(0, N_CHUNK // 2, pair_body, 0)

    finish_chunk(1, samp0 + (N_CHUNK - 1) * SAMP_PER_CHUNK)
    pltpu.make_async_copy(idx_hbm.at[pl.ds(samp0, SAMP_PER_CHUNK)],
                          idx_v.at[0], sem_idx).wait()
    for b in range(2):
        tail = samp0 + (N_CHUNK - 2 + b) * SAMP_PER_CHUNK
        pltpu.make_async_copy(rows_v.at[b],
                              out_hbm.at[pl.ds(tail, SAMP_PER_CHUNK)],
                              sem_out[b]).wait()


def kernel(input_ids, table):
    return _emb_lookup(input_ids, table)
